# baseline (speedup 1.0000x reference)
import functools

import jax
import jax.numpy as jnp
from jax.experimental import pallas as pl
from jax.experimental.pallas import tpu as pltpu
from jax.experimental.pallas import tpu_sc as plsc

_BETA = 0.25
_K = 8192
_C = 256
_TB = 256
_KB = 512
_PASS_BOUNDS = (0, 2736, 5472, 8192)


def _argmin_body(xsq_ref, cbsq_ref, x_ref, cb_ref, ids_ref, loss_ref):
    i = pl.program_id(0)
    x = x_ref[...]
    xb = x.astype(jnp.bfloat16)
    xsq = xsq_ref[...]

    inf = jnp.full((_TB,), jnp.inf, jnp.float32)
    pv = [inf, inf, inf]
    pi = [jnp.zeros((_TB,), jnp.int32)] * 3

    for j in range(_K // _KB):
        k0 = j * _KB
        cb = cb_ref[k0:k0 + _KB, :]
        mm = jax.lax.dot_general(
            xb, cb.astype(jnp.bfloat16), (((1,), (1,)), ((), ())),
            preferred_element_type=jnp.float32)
        sq = cbsq_ref[0:1, k0:k0 + _KB] + xsq
        dist = sq - mm * 2.0
        iota = jax.lax.broadcasted_iota(jnp.int32, (_TB, _KB), 1) + k0

        for p in range(3):
            lo, hi = _PASS_BOUNDS[p], _PASS_BOUNDS[p + 1]
            if hi <= k0 or lo >= k0 + _KB:
                continue
            if lo <= k0 and hi >= k0 + _KB:
                d = dist
                sel = None
            else:
                sel = (iota >= lo) & (iota < hi)
                d = jnp.where(sel, dist, jnp.inf)
            vmin = jnp.min(d, axis=1, keepdims=True)
            hit = d == vmin
            li = jnp.min(jnp.where(hit, iota, _K), axis=1)
            v = vmin[:, 0]
            upd = v < pv[p]
            pi[p] = jnp.where(upd, li, pi[p])
            pv[p] = jnp.where(upd, v, pv[p])

    acc_v = jnp.full((_TB,), jnp.inf, jnp.float32)
    acc_e = jnp.full((_TB,), jnp.inf, jnp.float32)
    acc_i = jnp.zeros((_TB,), jnp.int32)
    for p in range(3):
        keep = (acc_v < pv[p]) | ((acc_v == pv[p]) & (acc_i < pi[p]))
        acc_i = jnp.where(keep, acc_i, pi[p])
        acc_e = jnp.where(keep, acc_e, pv[p])
        acc_v = jnp.where(keep, acc_v, pv[p]).astype(jnp.bfloat16).astype(jnp.float32)

    ids_ref[0, 0, :] = acc_i

    @pl.when(i == 0)
    def _():
        loss_ref[...] = jnp.zeros_like(loss_ref)

    loss_ref[...] += jnp.sum(acc_e).reshape(1, 1)


def _argmin_call(x_, x_sq, cb, cb_sq):
    n_tok = x_.shape[0]
    n_blk = n_tok // _TB
    return pl.pallas_call(
        _argmin_body,
        grid=(n_blk,),
        in_specs=[
            pl.BlockSpec((_TB, 1), lambda i: (i, 0)),
            pl.BlockSpec((1, _K), lambda i: (0, 0)),
            pl.BlockSpec((_TB, _C), lambda i: (i, 0)),
            pl.BlockSpec((_K, _C), lambda i: (0, 0)),
        ],
        out_specs=[
            pl.BlockSpec((1, 1, _TB), lambda i: (i, 0, 0)),
            pl.BlockSpec((1, 1), lambda i: (0, 0)),
        ],
        out_shape=[
            jax.ShapeDtypeStruct((n_blk, 1, _TB), jnp.int32),
            jax.ShapeDtypeStruct((1, 1), jnp.float32),
        ],
    )(x_sq.reshape(n_blk * _TB, 1), cb_sq.reshape(1, _K), x_, cb)


def _gather_call(cb, ids_flat):
    info = plsc.get_sparse_core_info()
    nw = info.num_cores * info.num_subcores
    b = ids_flat.shape[0]
    b_per_w = b // nw
    ch = 128
    mesh = plsc.VectorSubcoreMesh(core_axis_name="c", subcore_axis_name="s")

    @functools.partial(
        pl.kernel, mesh=mesh,
        out_type=jax.ShapeDtypeStruct((b, _C), jnp.float32),
        scratch_types=[
            pltpu.VMEM((ch,), jnp.int32),
            pltpu.VMEM((ch, _C), jnp.float32),
            pltpu.SemaphoreType.DMA,
        ],
    )
    def k(table_hbm, idx_hbm, out_hbm, idx_v, rows_v, sem):
        wid = jax.lax.axis_index("s") * info.num_cores + jax.lax.axis_index("c")
        base = wid * b_per_w
        for j in range(b_per_w // ch):
            pltpu.sync_copy(idx_hbm.at[pl.ds(base + j * ch, ch)], idx_v)
            pltpu.async_copy(table_hbm.at[idx_v], rows_v, sem).wait()
            pltpu.sync_copy(rows_v, out_hbm.at[pl.ds(base + j * ch, ch)])

    return k(cb, ids_flat)


def kernel(x, code_book):
    b, c, h, w = x.shape
    x_ = jnp.transpose(x, (0, 2, 3, 1)).reshape(-1, c)
    x_sq = jnp.sum(x_ ** 2, axis=-1)
    cb_sq = jnp.sum(code_book ** 2, axis=-1)

    ids_blocks, loss_sum = _argmin_call(x_, x_sq, code_book, cb_sq)
    ids_flat = ids_blocks.reshape(-1)

    emb_flat = _gather_call(code_book, ids_flat)

    ids = ids_flat.reshape(b, h, w)
    emb = jnp.transpose(emb_flat.reshape(b, h, w, c), (0, 3, 1, 2))
    emb_loss = loss_sum[0, 0] * (1.0 + _BETA) / x.size
    return ids, emb, emb_loss


# speedup vs baseline: 1.0154x; 1.0154x over previous
import functools

import jax
import jax.numpy as jnp
from jax.experimental import pallas as pl
from jax.experimental.pallas import tpu as pltpu
from jax.experimental.pallas import tpu_sc as plsc

_BETA = 0.25
_K = 8192
_C = 256
_TB = 256
_KB = 512
_PASS_BOUNDS = (0, 2736, 5472, 8192)


def _argmin_body(xsq_ref, cbsq_ref, x_ref, cb_ref, ids_ref, loss_ref):
    i = pl.program_id(0)
    x = x_ref[...]
    xb = x.astype(jnp.bfloat16)
    xsq = xsq_ref[...]

    inf = jnp.full((_TB,), jnp.inf, jnp.float32)
    pv = [inf, inf, inf]
    pi = [jnp.zeros((_TB,), jnp.int32)] * 3
    iota_base = jax.lax.broadcasted_iota(jnp.int32, (_TB, _KB), 1)

    for j in range(_K // _KB):
        k0 = j * _KB
        cb = cb_ref[k0:k0 + _KB, :]
        mm = jax.lax.dot_general(
            xb, cb.astype(jnp.bfloat16), (((1,), (1,)), ((), ())),
            preferred_element_type=jnp.float32)
        sq = cbsq_ref[0:1, k0:k0 + _KB] + xsq
        dist = sq - mm * 2.0

        for p in range(3):
            lo, hi = _PASS_BOUNDS[p], _PASS_BOUNDS[p + 1]
            if hi <= k0 or lo >= k0 + _KB:
                continue
            if lo <= k0 and hi >= k0 + _KB:
                d = dist
            else:
                sel = (iota_base >= lo - k0) & (iota_base < hi - k0)
                d = jnp.where(sel, dist, jnp.inf)
            vmin = jnp.min(d, axis=1, keepdims=True)
            hit = d == vmin
            li = jnp.min(jnp.where(hit, iota_base, _K), axis=1) + k0
            v = vmin[:, 0]
            upd = v < pv[p]
            pi[p] = jnp.where(upd, li, pi[p])
            pv[p] = jnp.where(upd, v, pv[p])

    acc_v = jnp.full((_TB,), jnp.inf, jnp.float32)
    acc_e = jnp.full((_TB,), jnp.inf, jnp.float32)
    acc_i = jnp.zeros((_TB,), jnp.int32)
    for p in range(3):
        keep = (acc_v < pv[p]) | ((acc_v == pv[p]) & (acc_i < pi[p]))
        acc_i = jnp.where(keep, acc_i, pi[p])
        acc_e = jnp.where(keep, acc_e, pv[p])
        acc_v = jnp.where(keep, acc_v, pv[p]).astype(jnp.bfloat16).astype(jnp.float32)

    ids_ref[0, 0, :] = acc_i

    @pl.when(i == 0)
    def _():
        loss_ref[...] = jnp.zeros_like(loss_ref)

    loss_ref[...] += jnp.sum(acc_e).reshape(1, 1)


def _argmin_call(x_, x_sq, cb, cb_sq):
    n_tok = x_.shape[0]
    n_blk = n_tok // _TB
    return pl.pallas_call(
        _argmin_body,
        grid=(n_blk,),
        in_specs=[
            pl.BlockSpec((_TB, 1), lambda i: (i, 0)),
            pl.BlockSpec((1, _K), lambda i: (0, 0)),
            pl.BlockSpec((_TB, _C), lambda i: (i, 0)),
            pl.BlockSpec((_K, _C), lambda i: (0, 0)),
        ],
        out_specs=[
            pl.BlockSpec((1, 1, _TB), lambda i: (i, 0, 0)),
            pl.BlockSpec((1, 1), lambda i: (0, 0)),
        ],
        out_shape=[
            jax.ShapeDtypeStruct((n_blk, 1, _TB), jnp.int32),
            jax.ShapeDtypeStruct((1, 1), jnp.float32),
        ],
    )(x_sq.reshape(n_blk * _TB, 1), cb_sq.reshape(1, _K), x_, cb)


def _gather_call(cb, ids_flat):
    info = plsc.get_sparse_core_info()
    nw = info.num_cores * info.num_subcores
    b = ids_flat.shape[0]
    b_per_w = b // nw
    ch = 128
    mesh = plsc.VectorSubcoreMesh(core_axis_name="c", subcore_axis_name="s")

    @functools.partial(
        pl.kernel, mesh=mesh,
        out_type=jax.ShapeDtypeStruct((b, _C), jnp.float32),
        scratch_types=[
            pltpu.VMEM((ch,), jnp.int32),
            pltpu.VMEM((ch, _C), jnp.float32),
            pltpu.SemaphoreType.DMA,
        ],
    )
    def k(table_hbm, idx_hbm, out_hbm, idx_v, rows_v, sem):
        wid = jax.lax.axis_index("s") * info.num_cores + jax.lax.axis_index("c")
        base = wid * b_per_w
        for j in range(b_per_w // ch):
            pltpu.sync_copy(idx_hbm.at[pl.ds(base + j * ch, ch)], idx_v)
            pltpu.async_copy(table_hbm.at[idx_v], rows_v, sem).wait()
            pltpu.sync_copy(rows_v, out_hbm.at[pl.ds(base + j * ch, ch)])

    return k(cb, ids_flat)


def kernel(x, code_book):
    b, c, h, w = x.shape
    x_ = jnp.transpose(x, (0, 2, 3, 1)).reshape(-1, c)
    x_sq = jnp.sum(x_ ** 2, axis=-1)
    cb_sq = jnp.sum(code_book ** 2, axis=-1)

    ids_blocks, loss_sum = _argmin_call(x_, x_sq, code_book, cb_sq)
    ids_flat = ids_blocks.reshape(-1)

    emb_flat = _gather_call(code_book, ids_flat)

    ids = ids_flat.reshape(b, h, w)
    emb = jnp.transpose(emb_flat.reshape(b, h, w, c), (0, 3, 1, 2))
    emb_loss = loss_sum[0, 0] * (1.0 + _BETA) / x.size
    return ids, emb, emb_loss


# speedup vs baseline: 1.2109x; 1.1925x over previous
import functools

import jax
import jax.numpy as jnp
from jax.experimental import pallas as pl
from jax.experimental.pallas import tpu as pltpu
from jax.experimental.pallas import tpu_sc as plsc

_BETA = 0.25
_K = 8192
_C = 256
_TB = 512
_KB = 512
_PASS_BOUNDS = (0, 2736, 5472, 8192)


def _argmin_body(xsq_ref, cbsq_ref, x_ref, cb_ref, ids_ref, loss_ref):
    i = pl.program_id(0)
    x = x_ref[...]
    xb = x.astype(jnp.bfloat16)
    xsq = xsq_ref[...]

    inf = jnp.full((_TB,), jnp.inf, jnp.float32)
    pv = [inf, inf, inf]
    pi = [jnp.zeros((_TB,), jnp.int32)] * 3
    iota_base = jax.lax.broadcasted_iota(jnp.int32, (_TB, _KB), 1)

    for j in range(_K // _KB):
        k0 = j * _KB
        cb = cb_ref[k0:k0 + _KB, :]
        mm = jax.lax.dot_general(
            xb, cb.astype(jnp.bfloat16), (((1,), (1,)), ((), ())),
            preferred_element_type=jnp.float32)
        sq = cbsq_ref[0:1, k0:k0 + _KB] + xsq
        dist = sq - mm * 2.0

        for p in range(3):
            lo, hi = _PASS_BOUNDS[p], _PASS_BOUNDS[p + 1]
            if hi <= k0 or lo >= k0 + _KB:
                continue
            if lo <= k0 and hi >= k0 + _KB:
                d = dist
            else:
                sel = (iota_base >= lo - k0) & (iota_base < hi - k0)
                d = jnp.where(sel, dist, jnp.inf)
            vmin = jnp.min(d, axis=1, keepdims=True)
            hit = d == vmin
            li = jnp.min(jnp.where(hit, iota_base, _K), axis=1) + k0
            v = vmin[:, 0]
            upd = v < pv[p]
            pi[p] = jnp.where(upd, li, pi[p])
            pv[p] = jnp.where(upd, v, pv[p])

    acc_v = jnp.full((_TB,), jnp.inf, jnp.float32)
    acc_e = jnp.full((_TB,), jnp.inf, jnp.float32)
    acc_i = jnp.zeros((_TB,), jnp.int32)
    for p in range(3):
        keep = (acc_v < pv[p]) | ((acc_v == pv[p]) & (acc_i < pi[p]))
        acc_i = jnp.where(keep, acc_i, pi[p])
        acc_e = jnp.where(keep, acc_e, pv[p])
        acc_v = jnp.where(keep, acc_v, pv[p]).astype(jnp.bfloat16).astype(jnp.float32)

    ids_ref[0, 0, :] = acc_i

    @pl.when(i == 0)
    def _():
        loss_ref[...] = jnp.zeros_like(loss_ref)

    loss_ref[...] += jnp.sum(acc_e).reshape(1, 1)


def _argmin_call(x_, x_sq, cb, cb_sq):
    n_tok = x_.shape[0]
    n_blk = n_tok // _TB
    return pl.pallas_call(
        _argmin_body,
        grid=(n_blk,),
        in_specs=[
            pl.BlockSpec((_TB, 1), lambda i: (i, 0)),
            pl.BlockSpec((1, _K), lambda i: (0, 0)),
            pl.BlockSpec((_TB, _C), lambda i: (i, 0)),
            pl.BlockSpec((_K, _C), lambda i: (0, 0)),
        ],
        out_specs=[
            pl.BlockSpec((1, 1, _TB), lambda i: (i, 0, 0)),
            pl.BlockSpec((1, 1), lambda i: (0, 0)),
        ],
        out_shape=[
            jax.ShapeDtypeStruct((n_blk, 1, _TB), jnp.int32),
            jax.ShapeDtypeStruct((1, 1), jnp.float32),
        ],
    )(x_sq.reshape(n_blk * _TB, 1), cb_sq.reshape(1, _K), x_, cb)


def _gather_call(cb, ids_flat):
    info = plsc.get_sparse_core_info()
    nw = info.num_cores * info.num_subcores
    b = ids_flat.shape[0]
    b_per_w = b // nw
    mesh = plsc.VectorSubcoreMesh(core_axis_name="c", subcore_axis_name="s")
    ch = 256

    @functools.partial(
        pl.kernel, mesh=mesh,
        out_type=jax.ShapeDtypeStruct((b, _C), jnp.float32),
        scratch_types=[
            pltpu.VMEM((b_per_w,), jnp.int32),
            pltpu.VMEM((ch, _C), jnp.float32),
            pltpu.SemaphoreType.DMA,
        ],
    )
    def k(table_hbm, idx_hbm, out_hbm, idx_v, rows_v, sem):
        wid = jax.lax.axis_index("s") * info.num_cores + jax.lax.axis_index("c")
        base = wid * b_per_w
        pltpu.sync_copy(idx_hbm.at[pl.ds(base, b_per_w)], idx_v)
        for j in range(b_per_w // ch):
            pltpu.async_copy(table_hbm.at[idx_v.at[pl.ds(j * ch, ch)]],
                             rows_v, sem).wait()
            pltpu.sync_copy(rows_v, out_hbm.at[pl.ds(base + j * ch, ch)])

    return k(cb, ids_flat)


def kernel(x, code_book):
    b, c, h, w = x.shape
    x_ = jnp.transpose(x, (0, 2, 3, 1)).reshape(-1, c)
    x_sq = jnp.sum(x_ ** 2, axis=-1)
    cb_sq = jnp.sum(code_book ** 2, axis=-1)

    ids_blocks, loss_sum = _argmin_call(x_, x_sq, code_book, cb_sq)
    ids_flat = ids_blocks.reshape(-1)

    emb_flat = _gather_call(code_book, ids_flat)

    ids = ids_flat.reshape(b, h, w)
    emb = jnp.transpose(emb_flat.reshape(b, h, w, c), (0, 3, 1, 2))
    emb_loss = loss_sum[0, 0] * (1.0 + _BETA) / x.size
    return ids, emb, emb_loss


# speedup vs baseline: 1.2543x; 1.0358x over previous
import functools

import jax
import jax.numpy as jnp
from jax.experimental import pallas as pl
from jax.experimental.pallas import tpu as pltpu
from jax.experimental.pallas import tpu_sc as plsc

_BETA = 0.25
_K = 8192
_C = 256
_TB = 1024
_KB = 512
_PASS_BOUNDS = (0, 2736, 5472, 8192)


def _argmin_body(xsq_ref, cbsq_ref, x_ref, cb_ref, ids_ref, loss_ref):
    i = pl.program_id(0)
    x = x_ref[...]
    xb = x.astype(jnp.bfloat16)
    xsq = xsq_ref[...]

    inf = jnp.full((_TB,), jnp.inf, jnp.float32)
    pv = [inf, inf, inf]
    pi = [jnp.zeros((_TB,), jnp.int32)] * 3
    iota_base = jax.lax.broadcasted_iota(jnp.int32, (_TB, _KB), 1)

    for j in range(_K // _KB):
        k0 = j * _KB
        cb = cb_ref[k0:k0 + _KB, :]
        mm = jax.lax.dot_general(
            xb, cb.astype(jnp.bfloat16), (((1,), (1,)), ((), ())),
            preferred_element_type=jnp.float32)
        sq = cbsq_ref[0:1, k0:k0 + _KB] + xsq
        dist = sq - mm * 2.0

        for p in range(3):
            lo, hi = _PASS_BOUNDS[p], _PASS_BOUNDS[p + 1]
            if hi <= k0 or lo >= k0 + _KB:
                continue
            if lo <= k0 and hi >= k0 + _KB:
                d = dist
            else:
                sel = (iota_base >= lo - k0) & (iota_base < hi - k0)
                d = jnp.where(sel, dist, jnp.inf)
            vmin = jnp.min(d, axis=1, keepdims=True)
            hit = d == vmin
            li = jnp.min(jnp.where(hit, iota_base, _K), axis=1) + k0
            v = vmin[:, 0]
            upd = v < pv[p]
            pi[p] = jnp.where(upd, li, pi[p])
            pv[p] = jnp.where(upd, v, pv[p])

    acc_v = jnp.full((_TB,), jnp.inf, jnp.float32)
    acc_e = jnp.full((_TB,), jnp.inf, jnp.float32)
    acc_i = jnp.zeros((_TB,), jnp.int32)
    for p in range(3):
        keep = (acc_v < pv[p]) | ((acc_v == pv[p]) & (acc_i < pi[p]))
        acc_i = jnp.where(keep, acc_i, pi[p])
        acc_e = jnp.where(keep, acc_e, pv[p])
        acc_v = jnp.where(keep, acc_v, pv[p]).astype(jnp.bfloat16).astype(jnp.float32)

    ids_ref[0, 0, :] = acc_i

    @pl.when(i == 0)
    def _():
        loss_ref[...] = jnp.zeros_like(loss_ref)

    loss_ref[...] += jnp.sum(acc_e).reshape(1, 1)


def _argmin_call(x_, x_sq, cb, cb_sq):
    n_tok = x_.shape[0]
    n_blk = n_tok // _TB
    return pl.pallas_call(
        _argmin_body,
        grid=(n_blk,),
        in_specs=[
            pl.BlockSpec((_TB, 1), lambda i: (i, 0)),
            pl.BlockSpec((1, _K), lambda i: (0, 0)),
            pl.BlockSpec((_TB, _C), lambda i: (i, 0)),
            pl.BlockSpec((_K, _C), lambda i: (0, 0)),
        ],
        out_specs=[
            pl.BlockSpec((1, 1, _TB), lambda i: (i, 0, 0)),
            pl.BlockSpec((1, 1), lambda i: (0, 0)),
        ],
        out_shape=[
            jax.ShapeDtypeStruct((n_blk, 1, _TB), jnp.int32),
            jax.ShapeDtypeStruct((1, 1), jnp.float32),
        ],
    )(x_sq.reshape(n_blk * _TB, 1), cb_sq.reshape(1, _K), x_, cb)


def _gather_call(cb, ids_flat):
    info = plsc.get_sparse_core_info()
    nw = info.num_cores * info.num_subcores
    b = ids_flat.shape[0]
    b_per_w = b // nw
    mesh = plsc.VectorSubcoreMesh(core_axis_name="c", subcore_axis_name="s")
    ch = 256

    @functools.partial(
        pl.kernel, mesh=mesh,
        out_type=jax.ShapeDtypeStruct((b, _C), jnp.float32),
        scratch_types=[
            pltpu.VMEM((b_per_w,), jnp.int32),
            pltpu.VMEM((ch, _C), jnp.float32),
            pltpu.SemaphoreType.DMA,
        ],
    )
    def k(table_hbm, idx_hbm, out_hbm, idx_v, rows_v, sem):
        wid = jax.lax.axis_index("s") * info.num_cores + jax.lax.axis_index("c")
        base = wid * b_per_w
        pltpu.sync_copy(idx_hbm.at[pl.ds(base, b_per_w)], idx_v)
        for j in range(b_per_w // ch):
            pltpu.async_copy(table_hbm.at[idx_v.at[pl.ds(j * ch, ch)]],
                             rows_v, sem).wait()
            pltpu.sync_copy(rows_v, out_hbm.at[pl.ds(base + j * ch, ch)])

    return k(cb, ids_flat)


def kernel(x, code_book):
    b, c, h, w = x.shape
    x_ = jnp.transpose(x, (0, 2, 3, 1)).reshape(-1, c)
    x_sq = jnp.sum(x_ ** 2, axis=-1)
    cb_sq = jnp.sum(code_book ** 2, axis=-1)

    ids_blocks, loss_sum = _argmin_call(x_, x_sq, code_book, cb_sq)
    ids_flat = ids_blocks.reshape(-1)

    emb_flat = _gather_call(code_book, ids_flat)

    ids = ids_flat.reshape(b, h, w)
    emb = jnp.transpose(emb_flat.reshape(b, h, w, c), (0, 3, 1, 2))
    emb_loss = loss_sum[0, 0] * (1.0 + _BETA) / x.size
    return ids, emb, emb_loss
